# baseline (speedup 1.0000x reference)
import jax
import jax.numpy as jnp
from jax import lax
from jax.experimental import pallas as pl
from jax.experimental.pallas import tpu as pltpu
from jax.experimental.pallas import tpu_sc as plsc

T = 128
D = 1024
FF = 512
E = 64
K = 8
EB = 2
NS = E // EB

_info = plsc.get_sparse_core_info()
_NC = _info.num_cores
_NW = _NC * _info.num_subcores
_TPW = T // _NW
_APW = _TPW * K


def _sc_dispatch_body(idx_hbm, wts_hbm, coef_hbm, idx_v, wts_v, buf):
    wid = lax.axis_index("s") * _NC + lax.axis_index("c")
    base_t = wid * _TPW
    base_a = wid * _APW
    pltpu.sync_copy(idx_hbm.at[pl.ds(base_a, _APW)], idx_v)
    pltpu.sync_copy(wts_hbm.at[pl.ds(base_a, _APW)], wts_v)
    zero = jnp.zeros((16,), jnp.float32)
    lane16 = lax.iota(jnp.int32, 16)
    for r in range(_TPW):
        vi = idx_v[pl.ds((r // 2) * 16, 16)]
        vw = wts_v[pl.ds((r // 2) * 16, 16)]
        accs = [zero] * (E // 16)
        for k in range(K):
            lane = (r % 2) * K + k
            e_s = vi[lane]
            w_s = vw[lane]
            for c in range(E // 16):
                accs[c] = accs[c] + jnp.where(lane16 + c * 16 == e_s, w_s, 0.0)
        for c in range(E // 16):
            buf[pl.ds(r * E + c * 16, 16)] = accs[c]
    pltpu.sync_copy(buf, coef_hbm.at[pl.ds(base_t * E, _TPW * E)])


def _sc_dispatch(idx_flat, wts_flat):
    mesh = plsc.VectorSubcoreMesh(core_axis_name="c", subcore_axis_name="s")
    return pl.kernel(
        _sc_dispatch_body,
        out_type=jax.ShapeDtypeStruct((T * E,), jnp.float32),
        mesh=mesh,
        scratch_types=[
            pltpu.VMEM((_APW,), jnp.int32),
            pltpu.VMEM((_APW,), jnp.float32),
            pltpu.VMEM((_TPW * E,), jnp.float32),
        ],
    )(idx_flat, wts_flat)


def _moe_body(coef_ref, x_ref, w1_ref, w3_ref, w2_ref, out_ref):
    s = pl.program_id(0)
    x = x_ref[...]

    acc = jnp.zeros((T, D), jnp.float32)
    for j in range(EB):
        lane = lax.broadcasted_iota(jnp.int32, (T, E), 1)
        col = jnp.sum(jnp.where(lane == s * EB + j, coef_ref[...], 0.0),
                      axis=1, keepdims=True)
        w1 = w1_ref[j]
        w3 = w3_ref[j]
        w2 = w2_ref[j]
        g = lax.dot_general(x, w1, (((1,), (1,)), ((), ())),
                            preferred_element_type=jnp.float32)
        u = lax.dot_general(x, w3, (((1,), (1,)), ((), ())),
                            preferred_element_type=jnp.float32)
        h = jax.nn.gelu(g, approximate=True) * u
        y = lax.dot_general(h, w2, (((1,), (1,)), ((), ())),
                            preferred_element_type=jnp.float32)
        acc = acc + col * y

    @pl.when(s == 0)
    def _():
        out_ref[...] = acc

    @pl.when(s != 0)
    def _():
        out_ref[...] += acc


def kernel(hidden_states, top_k_index, top_k_weights, w1_weight, w2_weight, w3_weight):
    idx_flat = top_k_index.astype(jnp.int32).reshape(T * K)
    wts_flat = top_k_weights.reshape(T * K)
    coef = _sc_dispatch(idx_flat, wts_flat).reshape(T, E)
    return pl.pallas_call(
        _moe_body,
        grid=(NS,),
        in_specs=[
            pl.BlockSpec((T, E), lambda s: (0, 0)),
            pl.BlockSpec((T, D), lambda s: (0, 0)),
            pl.BlockSpec((EB, FF, D), lambda s: (s, 0, 0)),
            pl.BlockSpec((EB, FF, D), lambda s: (s, 0, 0)),
            pl.BlockSpec((EB, D, FF), lambda s: (s, 0, 0)),
        ],
        out_specs=pl.BlockSpec((T, D), lambda s: (0, 0)),
        out_shape=jax.ShapeDtypeStruct((T, D), jnp.float32),
    )(coef, hidden_states, w1_weight, w3_weight, w2_weight)
